# baseline (speedup 1.0000x reference)
import functools

import jax
import jax.numpy as jnp
from jax.experimental import pallas as pl
from jax.experimental.pallas import tpu as pltpu

_B = 128
_NLVL = 5


def _p8(c):
    return (c + 7) // 8 * 8


def _lsize(lvl):
    hw = (32 >> lvl) * (32 >> lvl)
    return _B * hw


def _qsplit(x, lvl):
    s = _lsize(lvl) // 4
    return [x[:, k * s:(k + 1) * s] for k in range(4)]


def _shift2d(x, lvl, dy, dx):
    if dy == 0 and dx == 0:
        return x
    c = x.shape[0]
    if lvl == _NLVL - 1:
        blocks = [x[:, k * _B:(k + 1) * _B] for k in range(4)]
        z = jnp.zeros((c, _B), x.dtype)
        out = []
        for y in range(2):
            for xx in range(2):
                sy, sx = y + dy, xx + dx
                ok = 0 <= sy < 2 and 0 <= sx < 2
                out.append(blocks[sy * 2 + sx] if ok else z)
        return jnp.concatenate(out, axis=1)
    blocks = _qsplit(x, lvl)
    out = []
    for py in range(2):
        for px in range(2):
            sy, sx = py + dy, px + dx
            qq = (sy & 1) * 2 + (sx & 1)
            out.append(_shift2d(blocks[qq], lvl + 1, sy >> 1, sx >> 1))
    return jnp.concatenate(out, axis=1)


_TAPS = [(dy, dx) for dy in (-1, 0, 1) for dx in (-1, 0, 1)]


def _conv3x3(groups, wcol, b, lvl, scr):
    cins = [g.shape[0] for g in groups]
    ktap = sum(cins)
    kk = 9 * ktap
    ll = groups[0].shape[1]
    if lvl <= 1:
        lq = ll // 4
        gblocks = [_qsplit(g, lvl) for g in groups]
        outs = []
        for py in range(2):
            for px in range(2):
                for ti, (dy, dx) in enumerate(_TAPS):
                    sy, sx = py + dy, px + dx
                    qq = (sy & 1) * 2 + (sx & 1)
                    r = ti * ktap
                    for gi in range(len(groups)):
                        scr[r:r + cins[gi], 0:lq] = _shift2d(
                            gblocks[gi][qq], lvl + 1, sy >> 1, sx >> 1)
                        r += cins[gi]
                outs.append(jnp.dot(wcol, scr[0:kk, 0:lq],
                                    preferred_element_type=jnp.float32))
        acc = jnp.concatenate(outs, axis=1)
    else:
        for ti, (dy, dx) in enumerate(_TAPS):
            r = ti * ktap
            for gi in range(len(groups)):
                scr[r:r + cins[gi], 0:ll] = _shift2d(groups[gi], lvl, dy, dx)
                r += cins[gi]
        acc = jnp.dot(wcol, scr[0:kk, 0:ll],
                      preferred_element_type=jnp.float32)
    return jnp.maximum(acc + b, 0.0)


def _pool(x, lvl):
    b0, b1, b2, b3 = _qsplit(x, lvl)
    return jnp.maximum(jnp.maximum(b0, b1), jnp.maximum(b2, b3))


def _deconv(hi, wdall, bd):
    y = jnp.dot(wdall, hi, preferred_element_type=jnp.float32)
    cp = wdall.shape[0] // 4
    return jnp.concatenate(
        [y[t * cp:(t + 1) * cp] for t in range(4)], axis=1) + bd


def _unet_kernel(x_ref,
                 w11, b11, w12, b12, w21, b21, w22, b22,
                 w31, b31, w32, b32, w41, b41, w42, b42,
                 wc1, bc1, wc2, bc2,
                 d4w, d4b, u41, u41b, u42, u42b,
                 d3w, d3b, u31, u31b, u32, u32b,
                 d2w, d2b, u21, u21b, u22, u22b,
                 d1w, d1b, u11, u11b, u12, u12b,
                 wf, bf, o_ref, s0, s1, s2, s3, s4):
    scr = {0: s0, 1: s1, 2: s2, 3: s3, 4: s4}
    x = x_ref[0]
    c1 = _conv3x3([_conv3x3([x], w11[...], b11[...], 0, s0)],
                  w12[...], b12[...], 0, s0)
    h = _pool(c1, 0)
    c2 = _conv3x3([_conv3x3([h], w21[...], b21[...], 1, s1)],
                  w22[...], b22[...], 1, s1)
    h = _pool(c2, 1)
    c3 = _conv3x3([_conv3x3([h], w31[...], b31[...], 2, s2)],
                  w32[...], b32[...], 2, s2)
    h = _pool(c3, 2)
    c4 = _conv3x3([_conv3x3([h], w41[...], b41[...], 3, s3)],
                  w42[...], b42[...], 3, s3)
    h = _pool(c4, 3)
    ce = _conv3x3([_conv3x3([h], wc1[...], bc1[...], 4, s4)],
                  wc2[...], bc2[...], 4, s4)

    def up(hi, skip, dw, db, w1, b1, w2, b2, lvl):
        cout = skip.shape[0]
        d = _deconv(hi, dw[...], db[...])[0:cout]
        hh = _conv3x3([d, skip], w1[...], b1[...], lvl, scr[lvl])
        return _conv3x3([hh], w2[...], b2[...], lvl, scr[lvl])

    u4 = up(ce, c4, d4w, d4b, u41, u41b, u42, u42b, 3)
    u3 = up(u4, c3, d3w, d3b, u31, u31b, u32, u32b, 2)
    u2 = up(u3, c2, d2w, d2b, u21, u21b, u22, u22b, 1)
    u1 = up(u2, c1, d1w, d1b, u11, u11b, u12, u12b, 0)

    logits = jnp.dot(wf[...], u1, preferred_element_type=jnp.float32) + bf[...]
    s = logits - jnp.max(logits, axis=0, keepdims=True)
    o_ref[0] = s - jnp.log(jnp.sum(jnp.exp(s), axis=0, keepdims=True))


def _k3(w):
    cout, cin = w.shape[0], w.shape[1]
    return jnp.transpose(w, (0, 2, 3, 1)).reshape(cout, 9 * cin)


def _kd(w):
    cin, cout = w.shape[0], w.shape[1]
    wt = jnp.transpose(w, (2, 3, 1, 0)).reshape(4, cout, cin)
    blks = [jnp.pad(wt[t], ((0, _p8(cout) - cout), (0, 0))) for t in range(4)]
    return jnp.concatenate(blks, axis=0)


def _col(b):
    return b.reshape(-1, 1)


def _colp(b):
    c = b.shape[0]
    return jnp.pad(b, (0, _p8(c) - c)).reshape(-1, 1)


_FWD_PERM = (0, 6, 11, 5, 10, 4, 9, 3, 8, 2, 7, 1)
_BWD_PERM = (0, 12, 1, 10, 8, 6, 4, 2, 11, 9, 7, 5, 3)


def kernel(x, conv1_c1_w, conv1_c1_b, conv1_c2_w, conv1_c2_b,
           conv2_c1_w, conv2_c1_b, conv2_c2_w, conv2_c2_b,
           conv3_c1_w, conv3_c1_b, conv3_c2_w, conv3_c2_b,
           conv4_c1_w, conv4_c1_b, conv4_c2_w, conv4_c2_b,
           center_c1_w, center_c1_b, center_c2_w, center_c2_b,
           up4_wd, up4_bd, up4_conv_c1_w, up4_conv_c1_b,
           up4_conv_c2_w, up4_conv_c2_b,
           up3_wd, up3_bd, up3_conv_c1_w, up3_conv_c1_b,
           up3_conv_c2_w, up3_conv_c2_b,
           up2_wd, up2_bd, up2_conv_c1_w, up2_conv_c1_b,
           up2_conv_c2_w, up2_conv_c2_b,
           up1_wd, up1_bd, up1_conv_c1_w, up1_conv_c1_b,
           up1_conv_c2_w, up1_conv_c2_b,
           final_w, final_b):
    n = x.shape[0]
    g = n // _B
    l0 = _lsize(0)

    t = x.reshape(g, _B, 2, 2, 2, 2, 2, 2, 2, 2, 2, 2)
    xb = t.transpose(*_FWD_PERM).reshape(g, 1, l0)

    args = [
        xb,
        _k3(conv1_c1_w), _col(conv1_c1_b), _k3(conv1_c2_w), _col(conv1_c2_b),
        _k3(conv2_c1_w), _col(conv2_c1_b), _k3(conv2_c2_w), _col(conv2_c2_b),
        _k3(conv3_c1_w), _col(conv3_c1_b), _k3(conv3_c2_w), _col(conv3_c2_b),
        _k3(conv4_c1_w), _col(conv4_c1_b), _k3(conv4_c2_w), _col(conv4_c2_b),
        _k3(center_c1_w), _col(center_c1_b),
        _k3(center_c2_w), _col(center_c2_b),
        _kd(up4_wd), _colp(up4_bd),
        _k3(up4_conv_c1_w), _col(up4_conv_c1_b),
        _k3(up4_conv_c2_w), _col(up4_conv_c2_b),
        _kd(up3_wd), _colp(up3_bd),
        _k3(up3_conv_c1_w), _col(up3_conv_c1_b),
        _k3(up3_conv_c2_w), _col(up3_conv_c2_b),
        _kd(up2_wd), _colp(up2_bd),
        _k3(up2_conv_c1_w), _col(up2_conv_c1_b),
        _k3(up2_conv_c2_w), _col(up2_conv_c2_b),
        _kd(up1_wd), _colp(up1_bd),
        _k3(up1_conv_c1_w), _col(up1_conv_c1_b),
        _k3(up1_conv_c2_w), _col(up1_conv_c2_b),
        final_w[:, :, 0, 0], _col(final_b),
    ]

    in_specs = [pl.BlockSpec((1, 1, l0), lambda i: (i, 0, 0))]
    for a in args[1:]:
        in_specs.append(
            pl.BlockSpec(a.shape, functools.partial(
                lambda nd, i: (0,) * nd, a.ndim)))

    out = pl.pallas_call(
        _unet_kernel,
        out_shape=jax.ShapeDtypeStruct((g, 3, l0), jnp.float32),
        grid=(g,),
        in_specs=in_specs,
        out_specs=pl.BlockSpec((1, 3, l0), lambda i: (i, 0, 0)),
        scratch_shapes=[
            pltpu.VMEM((72, _lsize(0) // 4), jnp.float32),
            pltpu.VMEM((144, _lsize(1) // 4), jnp.float32),
            pltpu.VMEM((288, _lsize(2)), jnp.float32),
            pltpu.VMEM((576, _lsize(3)), jnp.float32),
            pltpu.VMEM((576, _lsize(4)), jnp.float32),
        ],
        compiler_params=pltpu.CompilerParams(
            dimension_semantics=("parallel",),
            vmem_limit_bytes=100 * 1024 * 1024),
    )(*args)

    t = out.reshape(g, 3, 2, 2, 2, 2, 2, 2, 2, 2, 2, 2, _B)
    logp = t.transpose(*_BWD_PERM).reshape(n, 3, 32, 32)
    return logp, logp, logp, logp


# speedup vs baseline: 1.3662x; 1.3662x over previous
import functools

import jax
import jax.numpy as jnp
from jax.experimental import pallas as pl
from jax.experimental.pallas import tpu as pltpu

_B = 128
_NLVL = 5


def _lsize(lvl):
    hw = (32 >> lvl) * (32 >> lvl)
    return _B * hw


def _qsplit(x, lvl):
    s = _lsize(lvl) // 4
    return [x[:, k * s:(k + 1) * s] for k in range(4)]


def _shift2d(x, lvl, dy, dx):
    if dy == 0 and dx == 0:
        return x
    c = x.shape[0]
    if lvl == _NLVL - 1:
        blocks = [x[:, k * _B:(k + 1) * _B] for k in range(4)]
        z = jnp.zeros((c, _B), x.dtype)
        out = []
        for y in range(2):
            for xx in range(2):
                sy, sx = y + dy, xx + dx
                ok = 0 <= sy < 2 and 0 <= sx < 2
                out.append(blocks[sy * 2 + sx] if ok else z)
        return jnp.concatenate(out, axis=1)
    blocks = _qsplit(x, lvl)
    out = []
    for py in range(2):
        for px in range(2):
            sy, sx = py + dy, px + dx
            qq = (sy & 1) * 2 + (sx & 1)
            out.append(_shift2d(blocks[qq], lvl + 1, sy >> 1, sx >> 1))
    return jnp.concatenate(out, axis=1)


_TAPS = [(dy, dx) for dy in (-1, 0, 1) for dx in (-1, 0, 1)]


def _conv3x3(x, wcol, b, lvl):
    if lvl <= 1:
        blocks = _qsplit(x, lvl)
        outs = []
        for py in range(2):
            for px in range(2):
                cols = []
                for dy, dx in _TAPS:
                    sy, sx = py + dy, px + dx
                    qq = (sy & 1) * 2 + (sx & 1)
                    cols.append(_shift2d(blocks[qq], lvl + 1, sy >> 1, sx >> 1))
                xcol = jnp.concatenate(cols, axis=0)
                outs.append(jnp.dot(wcol, xcol,
                                    preferred_element_type=jnp.float32))
        acc = jnp.concatenate(outs, axis=1)
    else:
        xcol = jnp.concatenate(
            [_shift2d(x, lvl, dy, dx) for dy, dx in _TAPS], axis=0)
        acc = jnp.dot(wcol, xcol, preferred_element_type=jnp.float32)
    return jnp.maximum(acc + b, 0.0).astype(jnp.bfloat16)


def _pool(x, lvl):
    b0, b1, b2, b3 = _qsplit(x, lvl)
    return jnp.maximum(jnp.maximum(b0, b1), jnp.maximum(b2, b3))


def _deconv(hi, wdall, bd):
    y = jnp.dot(wdall, hi, preferred_element_type=jnp.float32)
    cout = wdall.shape[0] // 4
    return (jnp.concatenate(
        [y[t * cout:(t + 1) * cout] for t in range(4)], axis=1)
        + bd).astype(jnp.bfloat16)


def _unet_kernel(x_ref,
                 w11, b11, w12, b12, w21, b21, w22, b22,
                 w31, b31, w32, b32, w41, b41, w42, b42,
                 wc1, bc1, wc2, bc2,
                 d4w, d4b, u41, u41b, u42, u42b,
                 d3w, d3b, u31, u31b, u32, u32b,
                 d2w, d2b, u21, u21b, u22, u22b,
                 d1w, d1b, u11, u11b, u12, u12b,
                 wf, bf, o_ref):
    x = x_ref[0]
    c1 = _conv3x3(_conv3x3(x, w11[...], b11[...], 0), w12[...], b12[...], 0)
    h = _pool(c1, 0)
    c2 = _conv3x3(_conv3x3(h, w21[...], b21[...], 1), w22[...], b22[...], 1)
    h = _pool(c2, 1)
    c3 = _conv3x3(_conv3x3(h, w31[...], b31[...], 2), w32[...], b32[...], 2)
    h = _pool(c3, 2)
    c4 = _conv3x3(_conv3x3(h, w41[...], b41[...], 3), w42[...], b42[...], 3)
    h = _pool(c4, 3)
    ce = _conv3x3(_conv3x3(h, wc1[...], bc1[...], 4), wc2[...], bc2[...], 4)

    def up(hi, skip, dw, db, w1, b1, w2, b2, lvl):
        d = _deconv(hi, dw[...], db[...])
        cat = jnp.concatenate([d, skip], axis=0)
        hh = _conv3x3(cat, w1[...], b1[...], lvl)
        return _conv3x3(hh, w2[...], b2[...], lvl)

    u4 = up(ce, c4, d4w, d4b, u41, u41b, u42, u42b, 3)
    u3 = up(u4, c3, d3w, d3b, u31, u31b, u32, u32b, 2)
    u2 = up(u3, c2, d2w, d2b, u21, u21b, u22, u22b, 1)
    u1 = up(u2, c1, d1w, d1b, u11, u11b, u12, u12b, 0)

    logits = jnp.dot(wf[...], u1, preferred_element_type=jnp.float32) + bf[...]
    s = logits - jnp.max(logits, axis=0, keepdims=True)
    o_ref[0] = s - jnp.log(jnp.sum(jnp.exp(s), axis=0, keepdims=True))


def _k3(w, dtype=jnp.bfloat16):
    cout, cin = w.shape[0], w.shape[1]
    return jnp.transpose(w, (0, 2, 3, 1)).reshape(cout, 9 * cin).astype(dtype)


def _kd(w):
    cin, cout = w.shape[0], w.shape[1]
    return (jnp.transpose(w, (2, 3, 1, 0)).reshape(4 * cout, cin)
            .astype(jnp.bfloat16))


def _col(b):
    return b.reshape(-1, 1)


_FWD_PERM = (0, 6, 11, 5, 10, 4, 9, 3, 8, 2, 7, 1)
_BWD_PERM = (0, 12, 1, 10, 8, 6, 4, 2, 11, 9, 7, 5, 3)


def kernel(x, conv1_c1_w, conv1_c1_b, conv1_c2_w, conv1_c2_b,
           conv2_c1_w, conv2_c1_b, conv2_c2_w, conv2_c2_b,
           conv3_c1_w, conv3_c1_b, conv3_c2_w, conv3_c2_b,
           conv4_c1_w, conv4_c1_b, conv4_c2_w, conv4_c2_b,
           center_c1_w, center_c1_b, center_c2_w, center_c2_b,
           up4_wd, up4_bd, up4_conv_c1_w, up4_conv_c1_b,
           up4_conv_c2_w, up4_conv_c2_b,
           up3_wd, up3_bd, up3_conv_c1_w, up3_conv_c1_b,
           up3_conv_c2_w, up3_conv_c2_b,
           up2_wd, up2_bd, up2_conv_c1_w, up2_conv_c1_b,
           up2_conv_c2_w, up2_conv_c2_b,
           up1_wd, up1_bd, up1_conv_c1_w, up1_conv_c1_b,
           up1_conv_c2_w, up1_conv_c2_b,
           final_w, final_b):
    n = x.shape[0]
    g = n // _B
    l0 = _lsize(0)

    t = x.reshape(g, _B, 2, 2, 2, 2, 2, 2, 2, 2, 2, 2)
    xb = t.transpose(*_FWD_PERM).reshape(g, 1, l0)

    args = [
        xb,
        _k3(conv1_c1_w, jnp.float32), _col(conv1_c1_b),
        _k3(conv1_c2_w), _col(conv1_c2_b),
        _k3(conv2_c1_w), _col(conv2_c1_b), _k3(conv2_c2_w), _col(conv2_c2_b),
        _k3(conv3_c1_w), _col(conv3_c1_b), _k3(conv3_c2_w), _col(conv3_c2_b),
        _k3(conv4_c1_w), _col(conv4_c1_b), _k3(conv4_c2_w), _col(conv4_c2_b),
        _k3(center_c1_w), _col(center_c1_b),
        _k3(center_c2_w), _col(center_c2_b),
        _kd(up4_wd), _col(up4_bd),
        _k3(up4_conv_c1_w), _col(up4_conv_c1_b),
        _k3(up4_conv_c2_w), _col(up4_conv_c2_b),
        _kd(up3_wd), _col(up3_bd),
        _k3(up3_conv_c1_w), _col(up3_conv_c1_b),
        _k3(up3_conv_c2_w), _col(up3_conv_c2_b),
        _kd(up2_wd), _col(up2_bd),
        _k3(up2_conv_c1_w), _col(up2_conv_c1_b),
        _k3(up2_conv_c2_w), _col(up2_conv_c2_b),
        _kd(up1_wd), _col(up1_bd),
        _k3(up1_conv_c1_w), _col(up1_conv_c1_b),
        _k3(up1_conv_c2_w), _col(up1_conv_c2_b),
        final_w[:, :, 0, 0].astype(jnp.bfloat16), _col(final_b),
    ]

    in_specs = [pl.BlockSpec((1, 1, l0), lambda i: (i, 0, 0))]
    for a in args[1:]:
        in_specs.append(
            pl.BlockSpec(a.shape, functools.partial(
                lambda nd, i: (0,) * nd, a.ndim)))

    out = pl.pallas_call(
        _unet_kernel,
        out_shape=jax.ShapeDtypeStruct((g, 3, l0), jnp.float32),
        grid=(g,),
        in_specs=in_specs,
        out_specs=pl.BlockSpec((1, 3, l0), lambda i: (i, 0, 0)),
        compiler_params=pltpu.CompilerParams(
            dimension_semantics=("parallel",),
            vmem_limit_bytes=100 * 1024 * 1024),
    )(*args)

    t = out.reshape(g, 3, 2, 2, 2, 2, 2, 2, 2, 2, 2, 2, _B)
    logp = t.transpose(*_BWD_PERM).reshape(n, 3, 32, 32)
    return logp, logp, logp, logp


# speedup vs baseline: 1.4408x; 1.0546x over previous
import functools

import jax
import jax.numpy as jnp
from jax.experimental import pallas as pl
from jax.experimental.pallas import tpu as pltpu

_B = 128
_NLVL = 5


def _lsize(lvl):
    hw = (32 >> lvl) * (32 >> lvl)
    return _B * hw


def _qsplit(x, lvl):
    s = _lsize(lvl) // 4
    return [x[:, k * s:(k + 1) * s] for k in range(4)]


def _shift2d(x, lvl, dy, dx):
    if dy == 0 and dx == 0:
        return x
    c = x.shape[0]
    if lvl == _NLVL - 1:
        blocks = [x[:, k * _B:(k + 1) * _B] for k in range(4)]
        z = jnp.zeros((c, _B), x.dtype)
        out = []
        for y in range(2):
            for xx in range(2):
                sy, sx = y + dy, xx + dx
                ok = 0 <= sy < 2 and 0 <= sx < 2
                out.append(blocks[sy * 2 + sx] if ok else z)
        return jnp.concatenate(out, axis=1)
    blocks = _qsplit(x, lvl)
    out = []
    for py in range(2):
        for px in range(2):
            sy, sx = py + dy, px + dx
            qq = (sy & 1) * 2 + (sx & 1)
            out.append(_shift2d(blocks[qq], lvl + 1, sy >> 1, sx >> 1))
    return jnp.concatenate(out, axis=1)


_TAPS = [(dy, dx) for dy in (-1, 0, 1) for dx in (-1, 0, 1)]


def _conv3x3(x, wcol, b, lvl):
    if lvl <= 1:
        blocks = _qsplit(x, lvl)
        outs = []
        for py in range(2):
            for px in range(2):
                cols = []
                for dy, dx in _TAPS:
                    sy, sx = py + dy, px + dx
                    qq = (sy & 1) * 2 + (sx & 1)
                    cols.append(_shift2d(blocks[qq], lvl + 1, sy >> 1, sx >> 1))
                xcol = jnp.concatenate(cols, axis=0)
                outs.append(jnp.dot(wcol, xcol,
                                    preferred_element_type=jnp.float32))
        acc = jnp.concatenate(outs, axis=1)
    else:
        xcol = jnp.concatenate(
            [_shift2d(x, lvl, dy, dx) for dy, dx in _TAPS], axis=0)
        acc = jnp.dot(wcol, xcol, preferred_element_type=jnp.float32)
    return jnp.maximum(acc + b, 0.0)


def _pool(x, lvl):
    b0, b1, b2, b3 = _qsplit(x, lvl)
    return jnp.maximum(jnp.maximum(b0, b1), jnp.maximum(b2, b3))


def _deconv(hi, wdall, bd):
    y = jnp.dot(wdall, hi, preferred_element_type=jnp.float32)
    cout = wdall.shape[0] // 4
    return jnp.concatenate(
        [y[t * cout:(t + 1) * cout] for t in range(4)], axis=1) + bd


def _unet_kernel(x_ref,
                 w11, b11, w12, b12, w21, b21, w22, b22,
                 w31, b31, w32, b32, w41, b41, w42, b42,
                 wc1, bc1, wc2, bc2,
                 d4w, d4b, u41, u41b, u42, u42b,
                 d3w, d3b, u31, u31b, u32, u32b,
                 d2w, d2b, u21, u21b, u22, u22b,
                 d1w, d1b, u11, u11b, u12, u12b,
                 wf, bf, o_ref):
    x = x_ref[0]
    c1 = _conv3x3(_conv3x3(x, w11[...], b11[...], 0), w12[...], b12[...], 0)
    h = _pool(c1, 0)
    c2 = _conv3x3(_conv3x3(h, w21[...], b21[...], 1), w22[...], b22[...], 1)
    h = _pool(c2, 1)
    c3 = _conv3x3(_conv3x3(h, w31[...], b31[...], 2), w32[...], b32[...], 2)
    h = _pool(c3, 2)
    c4 = _conv3x3(_conv3x3(h, w41[...], b41[...], 3), w42[...], b42[...], 3)
    h = _pool(c4, 3)
    ce = _conv3x3(_conv3x3(h, wc1[...], bc1[...], 4), wc2[...], bc2[...], 4)

    def up(hi, skip, dw, db, w1, b1, w2, b2, lvl):
        d = _deconv(hi, dw[...], db[...])
        cat = jnp.concatenate([d, skip], axis=0)
        hh = _conv3x3(cat, w1[...], b1[...], lvl)
        return _conv3x3(hh, w2[...], b2[...], lvl)

    u4 = up(ce, c4, d4w, d4b, u41, u41b, u42, u42b, 3)
    u3 = up(u4, c3, d3w, d3b, u31, u31b, u32, u32b, 2)
    u2 = up(u3, c2, d2w, d2b, u21, u21b, u22, u22b, 1)
    u1 = up(u2, c1, d1w, d1b, u11, u11b, u12, u12b, 0)

    logits = jnp.dot(wf[...], u1, preferred_element_type=jnp.float32) + bf[...]
    s = logits - jnp.max(logits, axis=0, keepdims=True)
    o_ref[0] = s - jnp.log(jnp.sum(jnp.exp(s), axis=0, keepdims=True))


def _k3(w):
    cout, cin = w.shape[0], w.shape[1]
    return jnp.transpose(w, (0, 2, 3, 1)).reshape(cout, 9 * cin)


def _kd(w):
    cin, cout = w.shape[0], w.shape[1]
    return jnp.transpose(w, (2, 3, 1, 0)).reshape(4 * cout, cin)


def _col(b):
    return b.reshape(-1, 1)


_FWD_PERM = (0, 6, 11, 5, 10, 4, 9, 3, 8, 2, 7, 1)
_BWD_PERM = (0, 12, 1, 10, 8, 6, 4, 2, 11, 9, 7, 5, 3)


def kernel(x, conv1_c1_w, conv1_c1_b, conv1_c2_w, conv1_c2_b,
           conv2_c1_w, conv2_c1_b, conv2_c2_w, conv2_c2_b,
           conv3_c1_w, conv3_c1_b, conv3_c2_w, conv3_c2_b,
           conv4_c1_w, conv4_c1_b, conv4_c2_w, conv4_c2_b,
           center_c1_w, center_c1_b, center_c2_w, center_c2_b,
           up4_wd, up4_bd, up4_conv_c1_w, up4_conv_c1_b,
           up4_conv_c2_w, up4_conv_c2_b,
           up3_wd, up3_bd, up3_conv_c1_w, up3_conv_c1_b,
           up3_conv_c2_w, up3_conv_c2_b,
           up2_wd, up2_bd, up2_conv_c1_w, up2_conv_c1_b,
           up2_conv_c2_w, up2_conv_c2_b,
           up1_wd, up1_bd, up1_conv_c1_w, up1_conv_c1_b,
           up1_conv_c2_w, up1_conv_c2_b,
           final_w, final_b):
    n = x.shape[0]
    g = n // _B
    l0 = _lsize(0)

    t = x.reshape(g, _B, 32, 32).transpose(0, 2, 3, 1)
    t = jax.lax.optimization_barrier(t)
    t = t.reshape(g, 2, 2, 2, 2, 2, 2, 2, 2, 2, 2, _B)
    xb = t.transpose(0, 5, 10, 4, 9, 3, 8, 2, 7, 1, 6, 11).reshape(g, 1, l0)

    args = [
        xb,
        _k3(conv1_c1_w), _col(conv1_c1_b), _k3(conv1_c2_w), _col(conv1_c2_b),
        _k3(conv2_c1_w), _col(conv2_c1_b), _k3(conv2_c2_w), _col(conv2_c2_b),
        _k3(conv3_c1_w), _col(conv3_c1_b), _k3(conv3_c2_w), _col(conv3_c2_b),
        _k3(conv4_c1_w), _col(conv4_c1_b), _k3(conv4_c2_w), _col(conv4_c2_b),
        _k3(center_c1_w), _col(center_c1_b),
        _k3(center_c2_w), _col(center_c2_b),
        _kd(up4_wd), _col(up4_bd),
        _k3(up4_conv_c1_w), _col(up4_conv_c1_b),
        _k3(up4_conv_c2_w), _col(up4_conv_c2_b),
        _kd(up3_wd), _col(up3_bd),
        _k3(up3_conv_c1_w), _col(up3_conv_c1_b),
        _k3(up3_conv_c2_w), _col(up3_conv_c2_b),
        _kd(up2_wd), _col(up2_bd),
        _k3(up2_conv_c1_w), _col(up2_conv_c1_b),
        _k3(up2_conv_c2_w), _col(up2_conv_c2_b),
        _kd(up1_wd), _col(up1_bd),
        _k3(up1_conv_c1_w), _col(up1_conv_c1_b),
        _k3(up1_conv_c2_w), _col(up1_conv_c2_b),
        final_w[:, :, 0, 0], _col(final_b),
    ]

    in_specs = [pl.BlockSpec((1, 1, l0), lambda i: (i, 0, 0))]
    for a in args[1:]:
        in_specs.append(
            pl.BlockSpec(a.shape, functools.partial(
                lambda nd, i: (0,) * nd, a.ndim)))

    out = pl.pallas_call(
        _unet_kernel,
        out_shape=jax.ShapeDtypeStruct((g, 3, l0), jnp.float32),
        grid=(g,),
        in_specs=in_specs,
        out_specs=pl.BlockSpec((1, 3, l0), lambda i: (i, 0, 0)),
        compiler_params=pltpu.CompilerParams(
            dimension_semantics=("parallel",),
            vmem_limit_bytes=100 * 1024 * 1024),
    )(*args)

    t = out.reshape(g, 3, 2, 2, 2, 2, 2, 2, 2, 2, 2, 2, _B)
    logp = t.transpose(*_BWD_PERM).reshape(n, 3, 32, 32)
    return logp, logp, logp, logp
